# Initial kernel scaffold; baseline (speedup 1.0000x reference)
import jax
import jax.numpy as jnp
from jax.experimental import pallas as pl


def kernel(idx, W):
    raise NotImplementedError("write your pallas kernel here")


# speedup vs baseline: 2.9610x; 2.9610x over previous
import functools

import jax
import jax.numpy as jnp
from jax import lax
from jax.experimental import pallas as pl
from jax.experimental.pallas import tpu as pltpu
from jax.experimental.pallas import tpu_sc as plsc

_D = 128
_NC = 2
_NS = 16
_NW = _NC * _NS
_CH = 128


def _emb_body(idx_hbm, w_hbm, out_hbm, idx_v, rows_v, sem):
    nch = idx_v.shape[0]
    wid = lax.axis_index("s") * _NC + lax.axis_index("c")
    pltpu.sync_copy(idx_hbm.at[wid], idx_v)
    base = wid * (nch * _CH)

    def chunk(j, carry):
        pltpu.async_copy(w_hbm.at[idx_v.at[j]], rows_v, sem).wait()
        pltpu.sync_copy(rows_v, out_hbm.at[pl.ds(base + j * _CH, _CH)])
        return carry

    lax.fori_loop(0, nch, chunk, 0)


@jax.jit
def _emb(idx2, w):
    nrows = idx2.shape[0] * idx2.shape[1] * _CH
    nch = idx2.shape[1]
    mesh = plsc.VectorSubcoreMesh(core_axis_name="c", subcore_axis_name="s")
    f = pl.kernel(
        _emb_body,
        out_type=jax.ShapeDtypeStruct((nrows, _D), jnp.float32),
        mesh=mesh,
        scratch_types=[
            pltpu.VMEM((nch, _CH), jnp.int32),
            pltpu.VMEM((_CH, _D), jnp.float32),
            pltpu.SemaphoreType.DMA,
        ],
    )
    return f(idx2, w)


def kernel(idx, W):
    b, h = idx.shape
    n = b * h
    idx2 = idx.astype(jnp.int32).reshape(_NW, n // (_NW * _CH), _CH)
    out = _emb(idx2, W)
    return out.reshape(b, h, _D)


# speedup vs baseline: 3.3024x; 1.1153x over previous
import jax
import jax.numpy as jnp
from jax import lax
from jax.experimental import pallas as pl
from jax.experimental.pallas import tpu as pltpu
from jax.experimental.pallas import tpu_sc as plsc

_D = 128
_NC = 2
_NS = 16
_NW = _NC * _NS
_CH = 128
_NBUF = 5


def _emb_body(idx_hbm, w_hbm, out_hbm, idx_v, rows_v, gsem, ssem):
    nch = idx_v.shape[0]
    ng = nch // _NBUF
    wid = lax.axis_index("s") * _NC + lax.axis_index("c")
    pltpu.sync_copy(idx_hbm.at[wid], idx_v)
    base = wid * (nch * _CH)

    def gather(j, b):
        pltpu.async_copy(w_hbm.at[idx_v.at[j]], rows_v.at[b], gsem.at[b])

    def wait_gather(b):
        pltpu.make_async_copy(w_hbm.at[idx_v.at[0]], rows_v.at[b],
                              gsem.at[b]).wait()

    def store(j, b):
        pltpu.async_copy(rows_v.at[b], out_hbm.at[pl.ds(base + j * _CH, _CH)],
                         ssem.at[b])

    def wait_store(b):
        pltpu.make_async_copy(rows_v.at[b],
                              out_hbm.at[pl.ds(base, _CH)], ssem.at[b]).wait()

    for b in range(_NBUF):
        gather(b, b)

    def outer(g, carry):
        for b in range(_NBUF):
            wait_gather(b)
            store(g * _NBUF + b, b)

        @pl.when(g < ng - 1)
        def _():
            for b in range(_NBUF):
                wait_store(b)
                gather((g + 1) * _NBUF + b, b)

        return carry

    lax.fori_loop(0, ng, outer, 0)
    for b in range(_NBUF):
        wait_store(b)


@jax.jit
def _emb(idx3, w):
    nch = idx3.shape[1]
    nrows = _NW * nch * _CH
    mesh = plsc.VectorSubcoreMesh(core_axis_name="c", subcore_axis_name="s")
    f = pl.kernel(
        _emb_body,
        out_type=jax.ShapeDtypeStruct((nrows, _D), jnp.float32),
        mesh=mesh,
        scratch_types=[
            pltpu.VMEM((nch, _CH), jnp.int32),
            pltpu.VMEM((_NBUF, _CH, _D), jnp.float32),
            pltpu.SemaphoreType.DMA((_NBUF,)),
            pltpu.SemaphoreType.DMA((_NBUF,)),
        ],
    )
    return f(idx3, w)


def kernel(idx, W):
    b, h = idx.shape
    n = b * h
    idx3 = idx.astype(jnp.int32).reshape(_NW, n // (_NW * _CH), _CH)
    out = _emb(idx3, W)
    return out.reshape(b, h, _D)


# speedup vs baseline: 5.8767x; 1.7795x over previous
import functools

import jax
import jax.numpy as jnp
from jax import lax
from jax.experimental import pallas as pl
from jax.experimental.pallas import tpu as pltpu
from jax.experimental.pallas import tpu_sc as plsc

_D = 128
_NC = 2
_NS = 16
_NW = _NC * _NS
_G = 8
_HPAD = 64


def _emb_body(idx_hbm, w_hbm, out_hbm, idx_v, rows_v, gsem, ssem):
    npe = idx_v.shape[0]
    hist = out_hbm.shape[1]
    ngrp = npe // _G
    wid = lax.axis_index("s") * _NC + lax.axis_index("c")
    pltpu.sync_copy(idx_hbm.at[wid], idx_v)
    ebase = wid * npe

    def fire(grp, h):
        for b in range(_G):
            pltpu.async_copy(
                w_hbm.at[idx_v.at[grp * _G + b, pl.ds(0, hist)]],
                rows_v.at[h, b], gsem.at[h])

    def drain_gathers(h):
        for b in range(_G):
            pltpu.make_async_copy(w_hbm.at[idx_v.at[0, pl.ds(0, hist)]],
                                  rows_v.at[h, b], gsem.at[h]).wait()

    def store(grp, h):
        pltpu.async_copy(rows_v.at[h],
                         out_hbm.at[pl.ds(ebase + grp * _G, _G)], ssem.at[h])

    def wait_store(h):
        pltpu.make_async_copy(rows_v.at[h],
                              out_hbm.at[pl.ds(ebase, _G)], ssem.at[h]).wait()

    fire(0, 0)
    fire(1, 1)

    def outer(g2, carry):
        for h in range(2):
            grp = 2 * g2 + h
            drain_gathers(h)
            store(grp, h)

            @pl.when(g2 < ngrp // 2 - 1)
            def _():
                wait_store(h)
                fire(grp + 2, h)

        return carry

    lax.fori_loop(0, ngrp // 2, outer, 0)
    wait_store(0)
    wait_store(1)


@functools.partial(jax.jit, static_argnums=(2,))
def _emb(idx3, w, hist):
    batch = idx3.shape[0] * idx3.shape[1]
    mesh = plsc.VectorSubcoreMesh(core_axis_name="c", subcore_axis_name="s")
    f = pl.kernel(
        _emb_body,
        out_type=jax.ShapeDtypeStruct((batch, hist, _D), jnp.float32),
        mesh=mesh,
        scratch_types=[
            pltpu.VMEM((idx3.shape[1], _HPAD), jnp.int32),
            pltpu.VMEM((2, _G, hist, _D), jnp.float32),
            pltpu.SemaphoreType.DMA((2,)),
            pltpu.SemaphoreType.DMA((2,)),
        ],
    )
    return f(idx3, w)


def kernel(idx, W):
    b, h = idx.shape
    idx_p = jnp.zeros((b, _HPAD), jnp.int32).at[:, :h].set(idx.astype(jnp.int32))
    idx3 = idx_p.reshape(_NW, b // _NW, _HPAD)
    return _emb(idx3, W, h)


# speedup vs baseline: 10.1838x; 1.7329x over previous
import jax
import jax.numpy as jnp
from jax import lax
from jax.experimental import pallas as pl
from jax.experimental.pallas import tpu as pltpu
from jax.experimental.pallas import tpu_sc as plsc

_D = 128
_NC = 2
_NS = 16
_NW = _NC * _NS
_CH = 128
_NBUF = 5


def _emb_body(idx_hbm, w_hbm, out_hbm, idx_v, rows_v, gsem, ssem):
    nch = idx_v.shape[0]
    ng = nch // _NBUF
    wid = lax.axis_index("s") * _NC + lax.axis_index("c")
    pltpu.sync_copy(idx_hbm.at[wid], idx_v)
    base = wid * (nch * _CH)

    def gather(j, b):
        pltpu.async_copy(w_hbm.at[idx_v.at[j]], rows_v.at[b], gsem.at[b])

    def wait_gather(b):
        pltpu.make_async_copy(w_hbm.at[idx_v.at[0]], rows_v.at[b],
                              gsem.at[b]).wait()

    def store(j, b):
        pltpu.async_copy(rows_v.at[b], out_hbm.at[pl.ds(base + j * _CH, _CH)],
                         ssem.at[b])

    def wait_store(b):
        pltpu.make_async_copy(rows_v.at[b],
                              out_hbm.at[pl.ds(base, _CH)], ssem.at[b]).wait()

    for b in range(_NBUF):
        gather(b, b)

    def outer(g, carry):
        for b in range(_NBUF):
            wait_gather(b)
            store(g * _NBUF + b, b)

        @pl.when(g < ng - 1)
        def _():
            for b in range(_NBUF):
                wait_store(b)
                gather((g + 1) * _NBUF + b, b)

        return carry

    lax.fori_loop(0, ng, outer, 0)
    for b in range(_NBUF):
        wait_store(b)


@jax.jit
def _emb(idx3, w):
    nch = idx3.shape[1]
    nrows = _NW * nch * _CH
    mesh = plsc.VectorSubcoreMesh(core_axis_name="c", subcore_axis_name="s")
    f = pl.kernel(
        _emb_body,
        out_type=jax.ShapeDtypeStruct((nrows, _D), jnp.float32),
        mesh=mesh,
        scratch_types=[
            pltpu.VMEM((nch, _CH), jnp.int32),
            pltpu.VMEM((_NBUF, _CH, _D), jnp.float32),
            pltpu.SemaphoreType.DMA((_NBUF,)),
            pltpu.SemaphoreType.DMA((_NBUF,)),
        ],
    )
    return f(idx3, w)


def kernel(idx, W):
    b, h = idx.shape
    n = b * h
    idx_t = jnp.transpose(idx.astype(jnp.int32))
    idx3 = idx_t.reshape(_NW, n // (_NW * _CH), _CH)
    out = _emb(idx3, W)
    return jnp.transpose(out.reshape(h, b, _D), (1, 0, 2))


# speedup vs baseline: 10.5382x; 1.0348x over previous
import jax
import jax.numpy as jnp
from jax import lax
from jax.experimental import pallas as pl
from jax.experimental.pallas import tpu as pltpu
from jax.experimental.pallas import tpu_sc as plsc

_D = 128
_NC = 2
_NS = 16
_NW = _NC * _NS
_CH = 128
_NBUF = 5


def _emb_body(idx_hbm, w_hbm, out_hbm, idx_v, rows_v, gsem, ssem):
    nch = idx_v.shape[0]
    ng = nch // _NBUF
    wid = lax.axis_index("s") * _NC + lax.axis_index("c")
    pltpu.sync_copy(idx_hbm.at[wid], idx_v)
    base = wid * (nch * _CH)

    def gather(j, b):
        pltpu.async_copy(w_hbm.at[idx_v.at[j]], rows_v.at[b], gsem.at[b])

    def wait_gather(b):
        pltpu.make_async_copy(w_hbm.at[idx_v.at[0]], rows_v.at[b],
                              gsem.at[b]).wait()

    def store(j, b):
        pltpu.async_copy(rows_v.at[b], out_hbm.at[pl.ds(base + j * _CH, _CH)],
                         ssem.at[b])

    def wait_store(b):
        pltpu.make_async_copy(rows_v.at[b],
                              out_hbm.at[pl.ds(base, _CH)], ssem.at[b]).wait()

    gather(0, 0)

    def outer(g, carry):
        for b in range(_NBUF):
            j = g * _NBUF + b
            bp = (b - 1) % _NBUF
            bn = (b + 1) % _NBUF

            def do_store(jp=j - 1, bp=bp):
                wait_gather(bp)
                store(jp, bp)

            if b == 0:
                pl.when(g > 0)(do_store)
            else:
                do_store()

            if b == _NBUF - 1:
                def do_gather(jn=j + 1, bn=bn):
                    wait_store(bn)
                    gather(jn, bn)

                pl.when(g < ng - 1)(do_gather)
            else:
                def do_wait_store(bn=bn):
                    wait_store(bn)

                pl.when(g > 0)(do_wait_store)
                gather(j + 1, bn)

        return carry

    lax.fori_loop(0, ng, outer, 0)
    wait_gather((nch - 1) % _NBUF)
    store(nch - 1, (nch - 1) % _NBUF)
    for b in range(_NBUF):
        wait_store(b)


@jax.jit
def _emb(idx3, w):
    nch = idx3.shape[1]
    nrows = _NW * nch * _CH
    mesh = plsc.VectorSubcoreMesh(core_axis_name="c", subcore_axis_name="s")
    f = pl.kernel(
        _emb_body,
        out_type=jax.ShapeDtypeStruct((nrows, _D), jnp.float32),
        mesh=mesh,
        scratch_types=[
            pltpu.VMEM((nch, _CH), jnp.int32),
            pltpu.VMEM((_NBUF, _CH, _D), jnp.float32),
            pltpu.SemaphoreType.DMA((_NBUF,)),
            pltpu.SemaphoreType.DMA((_NBUF,)),
        ],
    )
    return f(idx3, w)


def kernel(idx, W):
    b, h = idx.shape
    n = b * h
    idx_t = jnp.transpose(idx.astype(jnp.int32))
    idx3 = idx_t.reshape(_NW, n // (_NW * _CH), _CH)
    out = _emb(idx3, W)
    return jnp.transpose(out.reshape(h, b, _D), (1, 0, 2))
